# baseline (speedup 1.0000x reference)
import functools

import jax
import jax.numpy as jnp
from jax import lax
from jax.experimental import pallas as pl
from jax.experimental.pallas import tpu as pltpu
from jax.experimental.pallas import tpu_sc as plsc

L = 16


def _build(B, R, N, CAP, b_per_w):
    N_PAD = ((N + L - 1) // L) * L
    CAP_PAD = ((CAP + L - 1) // L) * L
    mesh = plsc.VectorSubcoreMesh(core_axis_name="c", subcore_axis_name="s")

    @functools.partial(
        pl.kernel,
        mesh=mesh,
        out_type=jax.ShapeDtypeStruct((B,), jnp.float32),
        compiler_params=pltpu.CompilerParams(needs_layout_passes=False),
        scratch_types=[
            pltpu.VMEM((b_per_w,), jnp.int32),
            pltpu.VMEM((b_per_w,), jnp.int32),
            pltpu.VMEM((R, N), jnp.float32),
            pltpu.VMEM((R, N), jnp.float32),
            pltpu.VMEM((CAP_PAD,), jnp.int32),
            pltpu.VMEM((N_PAD,), jnp.int32),
            pltpu.VMEM((b_per_w,), jnp.float32),
            pltpu.SemaphoreType.DMA,
        ],
    )
    def demand_kernel(i_hbm, j_hbm, as_hbm, bs_hbm, zc_hbm, out_hbm,
                      i_v, j_v, as_v, bs_v, cap_v, mask_v, out_v, sem):
        wid = lax.axis_index("s") * 2 + lax.axis_index("c")
        base = wid * b_per_w

        copies = [
            pltpu.async_copy(zc_hbm.at[pl.ds(0, N_PAD)], mask_v, sem),
            pltpu.async_copy(zc_hbm.at[pl.ds(N_PAD, CAP_PAD)], cap_v, sem),
            pltpu.async_copy(i_hbm.at[pl.ds(base, b_per_w)], i_v, sem),
            pltpu.async_copy(j_hbm.at[pl.ds(base, b_per_w)], j_v, sem),
            pltpu.async_copy(as_hbm, as_v, sem),
            pltpu.async_copy(bs_hbm, bs_v, sem),
        ]

        zeros = jnp.zeros((L,), jnp.int32)
        ones = jnp.ones((L,), jnp.int32)

        for c in copies:
            c.wait()

        for k in range(CAP_PAD // L):
            idx = cap_v[pl.ds(k * L, L)]
            plsc.store_scatter(mask_v, [idx], ones)

        def body(k, carry):
            iv = i_v[pl.ds(k * L, L)]
            jv = j_v[pl.ds(k * L, L)]
            mi = plsc.load_gather(mask_v, [iv])
            mj = plsc.load_gather(mask_v, [jv])
            r = jnp.bitwise_or(mi, mj)
            ai = plsc.load_gather(as_v, [r, iv])
            aj = plsc.load_gather(as_v, [r, jv])
            bi = plsc.load_gather(bs_v, [r, iv])
            bj = plsc.load_gather(bs_v, [r, jv])
            out_v[pl.ds(k * L, L)] = ai * aj + bi + bj
            return carry
        lax.fori_loop(0, b_per_w // L, body, 0, unroll=1)

        pltpu.sync_copy(out_v, out_hbm.at[pl.ds(base, b_per_w)])

    return demand_kernel


def kernel(batch, As, Bs, capital_ids):
    B = batch.shape[0]
    R, N = As.shape
    CAP = capital_ids.shape[0]
    NW = 32
    b_per_w = B // NW

    N_PAD = ((N + L - 1) // L) * L
    CAP_PAD = ((CAP + L - 1) // L) * L
    cap32 = capital_ids.astype(jnp.int32)
    zc = jnp.concatenate([
        jnp.zeros((N_PAD,), jnp.int32),
        cap32,
        jnp.broadcast_to(cap32[0], (CAP_PAD - CAP,)),
    ])
    fn = _build(B, R, N, CAP, b_per_w)
    return fn(batch[:, 0], batch[:, 1], As, Bs, zc)


# speedup vs baseline: 1.0651x; 1.0651x over previous
import functools

import jax
import jax.numpy as jnp
from jax import lax
from jax.experimental import pallas as pl
from jax.experimental.pallas import tpu as pltpu
from jax.experimental.pallas import tpu_sc as plsc

L = 16


def _build(B, R, N, CAP, b_per_w):
    N_PAD = ((N + L - 1) // L) * L
    CAP_PAD = ((CAP + L - 1) // L) * L
    mesh = plsc.VectorSubcoreMesh(core_axis_name="c", subcore_axis_name="s")

    @functools.partial(
        pl.kernel,
        mesh=mesh,
        out_type=jax.ShapeDtypeStruct((B,), jnp.float32),
        compiler_params=pltpu.CompilerParams(needs_layout_passes=False),
        scratch_types=[
            pltpu.VMEM((b_per_w,), jnp.int32),
            pltpu.VMEM((b_per_w,), jnp.int32),
            pltpu.VMEM((R, N), jnp.float32),
            pltpu.VMEM((R, N), jnp.float32),
            pltpu.VMEM((CAP_PAD,), jnp.int32),
            pltpu.VMEM((N_PAD,), jnp.int32),
            pltpu.VMEM((b_per_w,), jnp.float32),
            pltpu.SemaphoreType.DMA,
        ],
    )
    def demand_kernel(i_hbm, j_hbm, as_hbm, bs_hbm, cap_hbm, out_hbm,
                      i_v, j_v, as_v, bs_v, cap_v, mask_v, out_v, sem):
        wid = lax.axis_index("s") * 2 + lax.axis_index("c")
        base = wid * b_per_w

        copies = [
            pltpu.async_copy(i_hbm.at[pl.ds(base, b_per_w)], i_v, sem),
            pltpu.async_copy(j_hbm.at[pl.ds(base, b_per_w)], j_v, sem),
            pltpu.async_copy(as_hbm, as_v, sem),
            pltpu.async_copy(bs_hbm, bs_v, sem),
            pltpu.async_copy(cap_hbm, cap_v.at[pl.ds(0, CAP)], sem),
        ]

        zeros = jnp.zeros((L,), jnp.int32)
        ones = jnp.ones((L,), jnp.int32)
        lane = jax.lax.iota(jnp.int32, L)

        def zero_body(k, carry):
            mask_v[pl.ds(k * L, L)] = zeros
            return carry
        lax.fori_loop(0, N_PAD // L, zero_body, 0, unroll=4)

        for c in copies:
            c.wait()

        for k in range(CAP_PAD // L):
            idx = cap_v[pl.ds(k * L, L)]
            if (k + 1) * L <= CAP:
                plsc.store_scatter(mask_v, [idx], ones)
            else:
                tail = jnp.full((L,), CAP - k * L, jnp.int32)
                plsc.store_scatter(mask_v, [idx], ones, mask=lane < tail)

        def body(k, carry):
            iv = i_v[pl.ds(k * L, L)]
            jv = j_v[pl.ds(k * L, L)]
            mi = plsc.load_gather(mask_v, [iv])
            mj = plsc.load_gather(mask_v, [jv])
            r = jnp.bitwise_or(mi, mj)
            ai = plsc.load_gather(as_v, [r, iv])
            aj = plsc.load_gather(as_v, [r, jv])
            bi = plsc.load_gather(bs_v, [r, iv])
            bj = plsc.load_gather(bs_v, [r, jv])
            out_v[pl.ds(k * L, L)] = ai * aj + bi + bj
            return carry
        lax.fori_loop(0, b_per_w // L, body, 0, unroll=1)

        pltpu.sync_copy(out_v, out_hbm.at[pl.ds(base, b_per_w)])

    return demand_kernel


def kernel(batch, As, Bs, capital_ids):
    B = batch.shape[0]
    R, N = As.shape
    CAP = capital_ids.shape[0]
    NW = 32
    b_per_w = B // NW

    fn = _build(B, R, N, CAP, b_per_w)
    return fn(batch[:, 0].astype(jnp.int32), batch[:, 1].astype(jnp.int32),
              As, Bs, capital_ids.astype(jnp.int32))


# speedup vs baseline: 1.0815x; 1.0153x over previous
import functools

import jax
import jax.numpy as jnp
from jax import lax
from jax.experimental import pallas as pl
from jax.experimental.pallas import tpu as pltpu
from jax.experimental.pallas import tpu_sc as plsc

L = 16


def _build(B, R, N, CAP, b_per_w):
    N_PAD = ((N + L - 1) // L) * L
    CAP_PAD = ((CAP + L - 1) // L) * L
    mesh = plsc.VectorSubcoreMesh(core_axis_name="c", subcore_axis_name="s")

    @functools.partial(
        pl.kernel,
        mesh=mesh,
        out_type=jax.ShapeDtypeStruct((B,), jnp.float32),
        compiler_params=pltpu.CompilerParams(needs_layout_passes=False),
        scratch_types=[
            pltpu.VMEM((b_per_w,), jnp.int32),
            pltpu.VMEM((b_per_w,), jnp.int32),
            pltpu.VMEM((R, N), jnp.float32),
            pltpu.VMEM((R, N), jnp.float32),
            pltpu.VMEM((CAP_PAD,), jnp.int32),
            pltpu.VMEM((N_PAD,), jnp.int32),
            pltpu.VMEM((b_per_w,), jnp.float32),
            pltpu.SemaphoreType.DMA,
        ],
    )
    def demand_kernel(i_hbm, j_hbm, as_hbm, bs_hbm, cap_hbm, out_hbm,
                      i_v, j_v, as_v, bs_v, cap_v, mask_v, out_v, sem):
        wid = lax.axis_index("s") * 2 + lax.axis_index("c")
        base = wid * b_per_w

        copies = [
            pltpu.async_copy(i_hbm.at[pl.ds(base, b_per_w)], i_v, sem),
            pltpu.async_copy(j_hbm.at[pl.ds(base, b_per_w)], j_v, sem),
            pltpu.async_copy(as_hbm, as_v, sem),
            pltpu.async_copy(bs_hbm, bs_v, sem),
            pltpu.async_copy(cap_hbm, cap_v.at[pl.ds(0, CAP)], sem),
        ]

        zeros = jnp.zeros((L,), jnp.int32)
        ones = jnp.ones((L,), jnp.int32)
        lane = jax.lax.iota(jnp.int32, L)

        def zero_body(k, carry):
            mask_v[pl.ds(k * L, L)] = zeros
            return carry
        lax.fori_loop(0, N_PAD // L, zero_body, 0, unroll=4)

        for c in copies:
            c.wait()

        for k in range(CAP_PAD // L):
            idx = cap_v[pl.ds(k * L, L)]
            if (k + 1) * L <= CAP:
                plsc.store_scatter(mask_v, [idx], ones)
            else:
                tail = jnp.full((L,), CAP - k * L, jnp.int32)
                plsc.store_scatter(mask_v, [idx], ones, mask=lane < tail)

        @plsc.parallel_loop(0, b_per_w, step=L, unroll=2)
        def body(k):
            iv = i_v[pl.ds(k, L)]
            jv = j_v[pl.ds(k, L)]
            mi = plsc.load_gather(mask_v, [iv])
            mj = plsc.load_gather(mask_v, [jv])
            r = jnp.bitwise_or(mi, mj)
            ai = plsc.load_gather(as_v, [r, iv])
            aj = plsc.load_gather(as_v, [r, jv])
            bi = plsc.load_gather(bs_v, [r, iv])
            bj = plsc.load_gather(bs_v, [r, jv])
            out_v[pl.ds(k, L)] = ai * aj + bi + bj

        pltpu.sync_copy(out_v, out_hbm.at[pl.ds(base, b_per_w)])

    return demand_kernel


def kernel(batch, As, Bs, capital_ids):
    B = batch.shape[0]
    R, N = As.shape
    CAP = capital_ids.shape[0]
    NW = 32
    b_per_w = B // NW

    fn = _build(B, R, N, CAP, b_per_w)
    return fn(batch[:, 0].astype(jnp.int32), batch[:, 1].astype(jnp.int32),
              As, Bs, capital_ids.astype(jnp.int32))
